# baseline (device time: 71036 ns/iter reference)
import jax
import jax.numpy as jnp
from jax import lax
from jax.experimental import pallas as pl
from jax.experimental.pallas import tpu as pltpu

N_DEV = 32
Q = 4


def kernel(x, Win0, Wout0, Win1, Wout1, Win2, Wout2):
    b, d = x.shape
    rows = b // N_DEV
    qc = d // Q

    def body(x_ref, win0, wout0, win1, wout1, win2, wout2, out_ref,
             part_ref, rs_buf, red_ref, xnext_ref,
             rs_send, rs_recv, ag_send, ag_recv, loc_rs, loc_ag):
        me = lax.axis_index("i")

        def rs_start(h, wout):
            groups = []
            for q in range(Q):
                cols = pl.ds(q * qc, qc)
                pq = jnp.dot(
                    h, wout[:, q * qc:(q + 1) * qc],
                    preferred_element_type=jnp.float32,
                )
                part_ref[:, cols] = pq
                sc = pltpu.make_async_copy(
                    part_ref.at[pl.ds(me * rows, rows), cols],
                    rs_buf.at[q, 0],
                    loc_rs.at[q],
                )
                sc.start()
                rds = []
                for off in range(1, N_DEV):
                    dst = lax.rem(me + off, N_DEV)
                    rdma = pltpu.make_async_remote_copy(
                        src_ref=part_ref.at[pl.ds(dst * rows, rows), cols],
                        dst_ref=rs_buf.at[q, off],
                        send_sem=rs_send.at[q, off],
                        recv_sem=rs_recv.at[q, off],
                        device_id=(dst,),
                        device_id_type=pl.DeviceIdType.MESH,
                    )
                    rdma.start()
                    rds.append(rdma)
                groups.append((sc, rds))
            return groups

        def rs_finish(groups, dst_ref):
            all_rds = []
            for q, (sc, rds) in enumerate(groups):
                sc.wait()
                acc = rs_buf[q, 0]
                for off, rdma in zip(range(1, N_DEV), rds):
                    rdma.wait_recv()
                    acc = acc + rs_buf[q, off]
                dst_ref[:, pl.ds(q * qc, qc)] = acc
                all_rds.extend(rds)
            return all_rds

        def ag_start():
            sc = pltpu.make_async_copy(
                red_ref, xnext_ref.at[pl.ds(me * rows, rows), :], loc_ag
            )
            sc.start()
            rds = []
            for off in range(1, N_DEV):
                dst = lax.rem(me + off, N_DEV)
                rdma = pltpu.make_async_remote_copy(
                    src_ref=red_ref,
                    dst_ref=xnext_ref.at[pl.ds(me * rows, rows), :],
                    send_sem=ag_send.at[off],
                    recv_sem=ag_recv.at[off],
                    device_id=(dst,),
                    device_id_type=pl.DeviceIdType.MESH,
                )
                rdma.start()
                rds.append(rdma)
            return sc, rds

        def ag_finish(sc, rds):
            sc.wait()
            for rdma in rds:
                rdma.wait_recv()
            return xnext_ref[...]

        def wait_sends(rds):
            for rdma in rds:
                rdma.wait_send()

        h = jnp.maximum(
            jnp.dot(x_ref[...], win0[...], preferred_element_type=jnp.float32),
            0.0,
        )
        rs0 = rs_start(h, wout0)
        rs0_rds = rs_finish(rs0, red_ref)
        ag0 = ag_start()
        xv = ag_finish(*ag0)

        h = jnp.maximum(
            jnp.dot(xv, win1[...], preferred_element_type=jnp.float32), 0.0
        )
        wait_sends(rs0_rds)
        rs1 = rs_start(h, wout1)
        wait_sends(ag0[1])
        rs1_rds = rs_finish(rs1, red_ref)
        ag1 = ag_start()
        xv = ag_finish(*ag1)

        h = jnp.maximum(
            jnp.dot(xv, win2[...], preferred_element_type=jnp.float32), 0.0
        )
        wait_sends(rs1_rds)
        rs2 = rs_start(h, wout2)
        rs2_rds = rs_finish(rs2, out_ref)
        wait_sends(ag1[1])
        wait_sends(rs2_rds)

    return pl.pallas_call(
        body,
        out_shape=jax.ShapeDtypeStruct((rows, d), jnp.float32),
        in_specs=[pl.BlockSpec(memory_space=pltpu.VMEM)] * 7,
        out_specs=pl.BlockSpec(memory_space=pltpu.VMEM),
        scratch_shapes=[
            pltpu.VMEM((b, d), jnp.float32),
            pltpu.VMEM((Q, N_DEV, rows, d // Q), jnp.float32),
            pltpu.VMEM((rows, d), jnp.float32),
            pltpu.VMEM((b, d), jnp.float32),
            pltpu.SemaphoreType.DMA((Q, N_DEV)),
            pltpu.SemaphoreType.DMA((Q, N_DEV)),
            pltpu.SemaphoreType.DMA((N_DEV,)),
            pltpu.SemaphoreType.DMA((N_DEV,)),
            pltpu.SemaphoreType.DMA((Q,)),
            pltpu.SemaphoreType.DMA,
        ],
        compiler_params=pltpu.CompilerParams(
            vmem_limit_bytes=100 * 1024 * 1024,
        ),
    )(x, Win0, Wout0, Win1, Wout1, Win2, Wout2)


# device time: 62615 ns/iter; 1.1345x vs baseline; 1.1345x over previous
import jax
import jax.numpy as jnp
from jax import lax
from jax.experimental import pallas as pl
from jax.experimental.pallas import tpu as pltpu

N_DEV = 32


def kernel(x, Win0, Wout0, Win1, Wout1, Win2, Wout2):
    b, d = x.shape
    rows = b // N_DEV

    def body(x_ref, win0, wout0, win1, wout1, win2, wout2, out_ref,
             part_ref, rs_buf, red_ref, xnext_ref,
             rs_send, rs_recv, ag_send, ag_recv, loc_sem):
        me = lax.axis_index("i")

        def reduce_scatter(part_val):
            part_ref[...] = part_val
            self_copy = pltpu.make_async_copy(
                part_ref.at[pl.ds(me * rows, rows), :], rs_buf.at[0], loc_sem
            )
            self_copy.start()
            rdmas = []
            for off in range(1, N_DEV):
                dst = lax.rem(me + off, N_DEV)
                rdma = pltpu.make_async_remote_copy(
                    src_ref=part_ref.at[pl.ds(dst * rows, rows), :],
                    dst_ref=rs_buf.at[off],
                    send_sem=rs_send.at[off],
                    recv_sem=rs_recv.at[off],
                    device_id=(dst,),
                    device_id_type=pl.DeviceIdType.MESH,
                )
                rdma.start()
                rdmas.append(rdma)
            self_copy.wait()
            acc = rs_buf[0].astype(jnp.float32)
            for off, rdma in zip(range(1, N_DEV), rdmas):
                rdma.wait_recv()
                acc = acc + rs_buf[off].astype(jnp.float32)
            return acc, rdmas

        def all_gather(red):
            red_ref[...] = red
            self_copy = pltpu.make_async_copy(
                red_ref, xnext_ref.at[pl.ds(me * rows, rows), :], loc_sem
            )
            self_copy.start()
            rdmas = []
            for off in range(1, N_DEV):
                dst = lax.rem(me + off, N_DEV)
                rdma = pltpu.make_async_remote_copy(
                    src_ref=red_ref,
                    dst_ref=xnext_ref.at[pl.ds(me * rows, rows), :],
                    send_sem=ag_send.at[off],
                    recv_sem=ag_recv.at[off],
                    device_id=(dst,),
                    device_id_type=pl.DeviceIdType.MESH,
                )
                rdma.start()
                rdmas.append(rdma)
            self_copy.wait()
            for rdma in rdmas:
                rdma.wait_recv()
            return xnext_ref[...], rdmas

        def wait_sends(rds):
            for rdma in rds:
                rdma.wait_send()

        def layer(xv16, win, wout):
            h = jnp.maximum(
                jnp.dot(xv16, win[...].astype(jnp.bfloat16),
                        preferred_element_type=jnp.float32),
                0.0,
            ).astype(jnp.bfloat16)
            p = jnp.dot(h, wout[...].astype(jnp.bfloat16),
                        preferred_element_type=jnp.float32)
            return p.astype(jnp.bfloat16)

        xv16 = x_ref[...].astype(jnp.bfloat16)
        red, rs0_rds = reduce_scatter(layer(xv16, win0, wout0))
        xv16, ag0_rds = all_gather(red.astype(jnp.bfloat16))

        p = layer(xv16, win1, wout1)
        wait_sends(rs0_rds)
        red, rs1_rds = reduce_scatter(p)
        wait_sends(ag0_rds)
        xv16, ag1_rds = all_gather(red.astype(jnp.bfloat16))

        p = layer(xv16, win2, wout2)
        wait_sends(rs1_rds)
        red, rs2_rds = reduce_scatter(p)
        out_ref[...] = red
        wait_sends(ag1_rds)
        wait_sends(rs2_rds)

    return pl.pallas_call(
        body,
        out_shape=jax.ShapeDtypeStruct((rows, d), jnp.float32),
        in_specs=[pl.BlockSpec(memory_space=pltpu.VMEM)] * 7,
        out_specs=pl.BlockSpec(memory_space=pltpu.VMEM),
        scratch_shapes=[
            pltpu.VMEM((b, d), jnp.bfloat16),
            pltpu.VMEM((N_DEV, rows, d), jnp.bfloat16),
            pltpu.VMEM((rows, d), jnp.bfloat16),
            pltpu.VMEM((b, d), jnp.bfloat16),
            pltpu.SemaphoreType.DMA((N_DEV,)),
            pltpu.SemaphoreType.DMA((N_DEV,)),
            pltpu.SemaphoreType.DMA((N_DEV,)),
            pltpu.SemaphoreType.DMA((N_DEV,)),
            pltpu.SemaphoreType.DMA,
        ],
        compiler_params=pltpu.CompilerParams(
            vmem_limit_bytes=100 * 1024 * 1024,
        ),
    )(x, Win0, Wout0, Win1, Wout1, Win2, Wout2)


# device time: 54960 ns/iter; 1.2925x vs baseline; 1.1393x over previous
import jax
import jax.numpy as jnp
from jax import lax
from jax.experimental import pallas as pl
from jax.experimental.pallas import tpu as pltpu

N_DEV = 32


def kernel(x, Win0, Wout0, Win1, Wout1, Win2, Wout2):
    b, d = x.shape
    rows = b // N_DEV

    def body(x_ref, win0, wout0, win1, wout1, win2, wout2, out_ref,
             part_ref, rs_buf, red_ref, xnext_ref,
             rs_send, rs_recv, ag_send, ag_recv, loc_sem):
        me = lax.axis_index("i")

        barrier_sem = pltpu.get_barrier_semaphore()
        for off in range(1, N_DEV):
            pl.semaphore_signal(
                barrier_sem, inc=1,
                device_id=(lax.rem(me + off, N_DEV),),
                device_id_type=pl.DeviceIdType.MESH,
            )

        def reduce_scatter(part_val):
            part_ref[...] = part_val
            self_copy = pltpu.make_async_copy(
                part_ref.at[pl.ds(me * rows, rows), :], rs_buf.at[0], loc_sem
            )
            self_copy.start()
            rdmas = []
            for off in range(1, N_DEV):
                dst = lax.rem(me + off, N_DEV)
                rdma = pltpu.make_async_remote_copy(
                    src_ref=part_ref.at[pl.ds(dst * rows, rows), :],
                    dst_ref=rs_buf.at[off],
                    send_sem=rs_send.at[off],
                    recv_sem=rs_recv.at[off],
                    device_id=(dst,),
                    device_id_type=pl.DeviceIdType.MESH,
                )
                rdma.start()
                rdmas.append(rdma)
            self_copy.wait()
            acc = rs_buf[0].astype(jnp.float32)
            for off, rdma in zip(range(1, N_DEV), rdmas):
                rdma.wait_recv()
                acc = acc + rs_buf[off].astype(jnp.float32)
            return acc, rdmas

        def all_gather(red):
            red_ref[...] = red
            self_copy = pltpu.make_async_copy(
                red_ref, xnext_ref.at[pl.ds(me * rows, rows), :], loc_sem
            )
            self_copy.start()
            rdmas = []
            for off in range(1, N_DEV):
                dst = lax.rem(me + off, N_DEV)
                rdma = pltpu.make_async_remote_copy(
                    src_ref=red_ref,
                    dst_ref=xnext_ref.at[pl.ds(me * rows, rows), :],
                    send_sem=ag_send.at[off],
                    recv_sem=ag_recv.at[off],
                    device_id=(dst,),
                    device_id_type=pl.DeviceIdType.MESH,
                )
                rdma.start()
                rdmas.append(rdma)
            self_copy.wait()
            for rdma in rdmas:
                rdma.wait_recv()
            return xnext_ref[...], rdmas

        def wait_sends(rds):
            for rdma in rds:
                rdma.wait_send()

        def layer(xv16, win, wout):
            h = jnp.maximum(
                jnp.dot(xv16, win[...].astype(jnp.bfloat16),
                        preferred_element_type=jnp.float32),
                0.0,
            ).astype(jnp.bfloat16)
            p = jnp.dot(h, wout[...].astype(jnp.bfloat16),
                        preferred_element_type=jnp.float32)
            return p.astype(jnp.bfloat16)

        xv16 = x_ref[...].astype(jnp.bfloat16)
        p = layer(xv16, win0, wout0)
        pl.semaphore_wait(barrier_sem, N_DEV - 1)
        red, rs0_rds = reduce_scatter(p)
        xv16, ag0_rds = all_gather(red.astype(jnp.bfloat16))

        p = layer(xv16, win1, wout1)
        wait_sends(rs0_rds)
        red, rs1_rds = reduce_scatter(p)
        wait_sends(ag0_rds)
        xv16, ag1_rds = all_gather(red.astype(jnp.bfloat16))

        p = layer(xv16, win2, wout2)
        wait_sends(rs1_rds)
        red, rs2_rds = reduce_scatter(p)
        out_ref[...] = red
        wait_sends(ag1_rds)
        wait_sends(rs2_rds)

    return pl.pallas_call(
        body,
        out_shape=jax.ShapeDtypeStruct((rows, d), jnp.float32),
        in_specs=[pl.BlockSpec(memory_space=pltpu.VMEM)] * 7,
        out_specs=pl.BlockSpec(memory_space=pltpu.VMEM),
        scratch_shapes=[
            pltpu.VMEM((b, d), jnp.bfloat16),
            pltpu.VMEM((N_DEV, rows, d), jnp.bfloat16),
            pltpu.VMEM((rows, d), jnp.bfloat16),
            pltpu.VMEM((b, d), jnp.bfloat16),
            pltpu.SemaphoreType.DMA((N_DEV,)),
            pltpu.SemaphoreType.DMA((N_DEV,)),
            pltpu.SemaphoreType.DMA((N_DEV,)),
            pltpu.SemaphoreType.DMA((N_DEV,)),
            pltpu.SemaphoreType.DMA,
        ],
        compiler_params=pltpu.CompilerParams(
            vmem_limit_bytes=100 * 1024 * 1024,
            collective_id=0,
        ),
    )(x, Win0, Wout0, Win1, Wout1, Win2, Wout2)
